# Initial kernel scaffold; baseline (speedup 1.0000x reference)
import jax
import jax.numpy as jnp
from jax.experimental import pallas as pl


def kernel(logits, S_rows, S_cols, S_vals):
    raise NotImplementedError("write your pallas kernel here")


# speedup vs baseline: 167.0971x; 167.0971x over previous
import functools

import jax
import jax.numpy as jnp
from jax import lax
from jax.experimental import pallas as pl
from jax.experimental.pallas import tpu as pltpu
from jax.experimental.pallas import tpu_sc as plsc

_V = 65536
_NNZ = 4194304
_ALPHA = 0.5
_NC = 2
_NS = 16
_NW = _NC * _NS
_NNZ_W = _NNZ // _NW
_CHUNK = 4096
_NCHUNK = _NNZ_W // _CHUNK
_GROUPS = _CHUNK // 16

_mesh = plsc.VectorSubcoreMesh(core_axis_name="c", subcore_axis_name="s")


@functools.partial(
    pl.kernel,
    out_type=jax.ShapeDtypeStruct((_NW, _V), jnp.float32),
    mesh=_mesh,
    scratch_types=[
        pltpu.VMEM((_V // 2,), jnp.int32),
        pltpu.VMEM((_V,), jnp.float32),
        pltpu.VMEM((_CHUNK,), jnp.int32),
        pltpu.VMEM((_CHUNK,), jnp.int32),
        pltpu.VMEM((_CHUNK,), jnp.float32),
    ],
    compiler_params=pltpu.CompilerParams(needs_layout_passes=False),
)
def _spmv_sc(packed_hbm, rows_hbm, cols_hbm, vals_hbm, out_hbm,
             plog, acc, rbuf, cbuf, vbuf):
    wid = lax.axis_index("s") * _NC + lax.axis_index("c")
    base = wid * _NNZ_W

    pltpu.sync_copy(packed_hbm, plog)

    zero = jnp.zeros((16,), jnp.float32)

    def _zero(i, c):
        acc[pl.ds(i * 16, 16)] = zero
        return c

    lax.fori_loop(0, _V // 16, _zero, 0)

    def _chunk(ci, c):
        off = base + ci * _CHUNK
        pltpu.sync_copy(rows_hbm.at[pl.ds(off, _CHUNK)], rbuf)
        pltpu.sync_copy(cols_hbm.at[pl.ds(off, _CHUNK)], cbuf)
        pltpu.sync_copy(vals_hbm.at[pl.ds(off, _CHUNK)], vbuf)

        def _grp(g, c2):
            s = g * 16
            c16 = cbuf[pl.ds(s, 16)]
            r16 = rbuf[pl.ds(s, 16)]
            v16 = vbuf[pl.ds(s, 16)]
            w = plsc.load_gather(plog, [lax.shift_right_logical(c16, 1)])
            hi = jnp.bitwise_and(w, jnp.int32(-65536))
            lo = lax.shift_left(w, 16)
            bits = jnp.where(jnp.bitwise_and(c16, 1) == 1, hi, lo)
            lg = plsc.bitcast(bits, jnp.float32)
            plsc.addupdate_scatter(acc, [r16], lg * v16)
            return c2

        lax.fori_loop(0, _GROUPS, _grp, 0)
        return c

    lax.fori_loop(0, _NCHUNK, _chunk, 0)
    pltpu.sync_copy(acc, out_hbm.at[wid])


def _combine_body(p_ref, l_ref, o_ref):
    o_ref[...] = jnp.sum(p_ref[...], axis=0) - _ALPHA * l_ref[...]


_combine = pl.pallas_call(
    _combine_body,
    out_shape=jax.ShapeDtypeStruct((512, 128), jnp.float32),
)


def kernel(logits, S_rows, S_cols, S_vals):
    packed = lax.bitcast_convert_type(
        logits.astype(jnp.bfloat16).reshape(_V // 2, 2), jnp.int32)
    partials = _spmv_sc(packed, S_rows, S_cols, S_vals)
    out = _combine(partials.reshape(_NW, 512, 128), logits.reshape(512, 128))
    return out.reshape(_V)


# speedup vs baseline: 229.7453x; 1.3749x over previous
import functools

import jax
import jax.numpy as jnp
from jax import lax
from jax.experimental import pallas as pl
from jax.experimental.pallas import tpu as pltpu
from jax.experimental.pallas import tpu_sc as plsc

_V = 65536
_NNZ = 4194304
_ALPHA = 0.5
_NC = 2
_NS = 16
_NW = _NC * _NS
_NNZ_W = _NNZ // _NW
_CHUNK = 4096
_NCHUNK = _NNZ_W // _CHUNK
_GROUPS = _CHUNK // 16
_UNROLL = 8

_mesh = plsc.VectorSubcoreMesh(core_axis_name="c", subcore_axis_name="s")


@functools.partial(
    pl.kernel,
    out_type=jax.ShapeDtypeStruct((_NW, _V), jnp.float32),
    mesh=_mesh,
    scratch_types=[
        pltpu.VMEM((_V // 2,), jnp.int32),
        pltpu.VMEM((_V,), jnp.float32),
        pltpu.VMEM((2, _CHUNK), jnp.int32),
        pltpu.VMEM((2, _CHUNK), jnp.int32),
        pltpu.VMEM((2, _CHUNK), jnp.float32),
        pltpu.SemaphoreType.DMA,
        pltpu.SemaphoreType.DMA,
    ],
    compiler_params=pltpu.CompilerParams(needs_layout_passes=False),
)
def _spmv_sc(packed_hbm, rows_hbm, cols_hbm, vals_hbm, out_hbm,
             plog, acc, rbuf, cbuf, vbuf, sem0, sem1):
    wid = lax.axis_index("s") * _NC + lax.axis_index("c")
    base = wid * _NNZ_W
    sems = (sem0, sem1)

    def _copies(ci, slot):
        off = base + ci * _CHUNK
        sem = sems[slot]
        return (
            pltpu.make_async_copy(rows_hbm.at[pl.ds(off, _CHUNK)],
                                  rbuf.at[slot], sem),
            pltpu.make_async_copy(cols_hbm.at[pl.ds(off, _CHUNK)],
                                  cbuf.at[slot], sem),
            pltpu.make_async_copy(vals_hbm.at[pl.ds(off, _CHUNK)],
                                  vbuf.at[slot], sem),
        )

    def _start(ci, slot):
        for d in _copies(ci, slot):
            d.start()

    def _wait(ci, slot):
        for d in _copies(ci, slot):
            d.wait()

    _start(0, 0)
    _start(1, 1)

    pltpu.sync_copy(packed_hbm, plog)

    zero = jnp.zeros((16,), jnp.float32)

    def _zero(i, c):
        s = i * (16 * 8)
        for k in range(8):
            acc[pl.ds(s + k * 16, 16)] = zero
        return c

    lax.fori_loop(0, _V // (16 * 8), _zero, 0)

    def _compute(slot):
        def _grp(g, c2):
            s0 = g * (16 * _UNROLL)
            for k in range(_UNROLL):
                s = s0 + k * 16
                c16 = cbuf[slot, pl.ds(s, 16)]
                r16 = rbuf[slot, pl.ds(s, 16)]
                v16 = vbuf[slot, pl.ds(s, 16)]
                w = plsc.load_gather(plog, [lax.shift_right_logical(c16, 1)])
                hi = jnp.bitwise_and(w, jnp.int32(-65536))
                lo = lax.shift_left(w, 16)
                bits = jnp.where(jnp.bitwise_and(c16, 1) == 1, hi, lo)
                lg = plsc.bitcast(bits, jnp.float32)
                plsc.addupdate_scatter(acc, [r16], lg * v16)
            return c2

        lax.fori_loop(0, _GROUPS // _UNROLL, _grp, 0)

    def _pair(pi, c):
        ci0 = pi * 2
        _wait(ci0, 0)
        _compute(0)

        @pl.when(ci0 + 2 < _NCHUNK)
        def _():
            _start(ci0 + 2, 0)

        _wait(ci0 + 1, 1)
        _compute(1)

        @pl.when(ci0 + 3 < _NCHUNK)
        def _():
            _start(ci0 + 3, 1)

        return c

    lax.fori_loop(0, _NCHUNK // 2, _pair, 0)
    pltpu.sync_copy(acc, out_hbm.at[wid])


def _combine_body(p_ref, l_ref, o_ref):
    o_ref[...] = jnp.sum(p_ref[...], axis=0) - _ALPHA * l_ref[...]


_combine = pl.pallas_call(
    _combine_body,
    out_shape=jax.ShapeDtypeStruct((512, 128), jnp.float32),
)


def kernel(logits, S_rows, S_cols, S_vals):
    packed = lax.bitcast_convert_type(
        logits.astype(jnp.bfloat16).reshape(_V // 2, 2), jnp.int32)
    partials = _spmv_sc(packed, S_rows, S_cols, S_vals)
    out = _combine(partials.reshape(_NW, 512, 128), logits.reshape(512, 128))
    return out.reshape(_V)


# speedup vs baseline: 381.7207x; 1.6615x over previous
import functools

import jax
import jax.numpy as jnp
from jax import lax
from jax.experimental import pallas as pl
from jax.experimental.pallas import tpu as pltpu
from jax.experimental.pallas import tpu_sc as plsc

_V = 65536
_NNZ = 4194304
_ALPHA = 0.5
_NC = 2
_NS = 16
_NW = _NC * _NS
_NNZ_W = _NNZ // _NW
_CHUNK = 4096
_NCHUNK = _NNZ_W // _CHUNK
_GROUPS = _CHUNK // 16
_UNROLL = 8

_mesh = plsc.VectorSubcoreMesh(core_axis_name="c", subcore_axis_name="s")


@functools.partial(
    pl.kernel,
    out_type=jax.ShapeDtypeStruct((_NW, _V), jnp.float32),
    mesh=_mesh,
    scratch_types=[
        pltpu.VMEM((_V // 2,), jnp.int32),
        pltpu.VMEM((_V,), jnp.float32),
        pltpu.VMEM((2, _CHUNK), jnp.int32),
        pltpu.VMEM((2, _CHUNK), jnp.int32),
        pltpu.VMEM((2, _CHUNK), jnp.float32),
        pltpu.SemaphoreType.DMA,
        pltpu.SemaphoreType.DMA,
    ],
    compiler_params=pltpu.CompilerParams(needs_layout_passes=False),
)
def _spmv_sc(packed_hbm, rows_hbm, cols_hbm, vals_hbm, out_hbm,
             plog, acc, rbuf, cbuf, vbuf, sem0, sem1):
    wid = lax.axis_index("s") * _NC + lax.axis_index("c")
    base = wid * _NNZ_W
    sems = (sem0, sem1)

    def _copies(ci, slot):
        off = base + ci * _CHUNK
        sem = sems[slot]
        return (
            pltpu.make_async_copy(rows_hbm.at[pl.ds(off, _CHUNK)],
                                  rbuf.at[slot], sem),
            pltpu.make_async_copy(cols_hbm.at[pl.ds(off, _CHUNK)],
                                  cbuf.at[slot], sem),
            pltpu.make_async_copy(vals_hbm.at[pl.ds(off, _CHUNK)],
                                  vbuf.at[slot], sem),
        )

    def _start(ci, slot):
        for d in _copies(ci, slot):
            d.start()

    def _wait(ci, slot):
        for d in _copies(ci, slot):
            d.wait()

    _start(0, 0)
    _start(1, 1)

    pltpu.sync_copy(packed_hbm, plog)

    zero = jnp.zeros((16,), jnp.float32)

    def _zero(i, c):
        s = i * (16 * 8)
        for k in range(8):
            acc[pl.ds(s + k * 16, 16)] = zero
        return c

    lax.fori_loop(0, _V // (16 * 8), _zero, 0)

    def _compute(slot):
        @plsc.parallel_loop(0, _GROUPS, 1, unroll=_UNROLL)
        def _grp(g):
            s = g * 16
            c16 = cbuf[slot, pl.ds(s, 16)]
            r16 = rbuf[slot, pl.ds(s, 16)]
            v16 = vbuf[slot, pl.ds(s, 16)]
            w = plsc.load_gather(plog, [lax.shift_right_logical(c16, 1)])
            hi = jnp.bitwise_and(w, jnp.int32(-65536))
            lo = lax.shift_left(w, 16)
            bits = jnp.where(jnp.bitwise_and(c16, 1) == 1, hi, lo)
            lg = plsc.bitcast(bits, jnp.float32)
            plsc.addupdate_scatter(acc, [r16], lg * v16)

    def _pair(pi, c):
        ci0 = pi * 2
        _wait(ci0, 0)
        _compute(0)

        @pl.when(ci0 + 2 < _NCHUNK)
        def _():
            _start(ci0 + 2, 0)

        _wait(ci0 + 1, 1)
        _compute(1)

        @pl.when(ci0 + 3 < _NCHUNK)
        def _():
            _start(ci0 + 3, 1)

        return c

    lax.fori_loop(0, _NCHUNK // 2, _pair, 0)
    pltpu.sync_copy(acc, out_hbm.at[wid])


def _combine_body(p_ref, l_ref, o_ref):
    o_ref[...] = jnp.sum(p_ref[...], axis=0) - _ALPHA * l_ref[...]


_combine = pl.pallas_call(
    _combine_body,
    out_shape=jax.ShapeDtypeStruct((512, 128), jnp.float32),
)


def kernel(logits, S_rows, S_cols, S_vals):
    packed = lax.bitcast_convert_type(
        logits.astype(jnp.bfloat16).reshape(_V // 2, 2), jnp.int32)
    partials = _spmv_sc(packed, S_rows, S_cols, S_vals)
    out = _combine(partials.reshape(_NW, 512, 128), logits.reshape(512, 128))
    return out.reshape(_V)
